# Initial kernel scaffold; baseline (speedup 1.0000x reference)
import jax
import jax.numpy as jnp
from jax.experimental import pallas as pl


def kernel(x, edge_index, edge_label_index, W1, b1, W2, b2):
    raise NotImplementedError("write your pallas kernel here")


# speedup vs baseline: 11.2920x; 11.2920x over previous
import functools

import jax
import jax.numpy as jnp
from jax import lax
from jax.experimental import pallas as pl
from jax.experimental.pallas import tpu as pltpu
from jax.experimental.pallas import tpu_sc as plsc

NC = 2
NS = 16
LANES = 16
NW = NC * NS

_MESH = plsc.VectorSubcoreMesh(core_axis_name="c", subcore_axis_name="s")


@functools.lru_cache(maxsize=None)
def _deg_kernel(E: int, N: int, D: int):
    per_w = E // NW
    C = 80
    n_ch = per_w // C
    rmain = (N // NS) // 8 * 8
    tbase = rmain * NS
    tail = N - tbase

    @functools.partial(
        pl.kernel,
        out_type=jax.ShapeDtypeStruct((NC, N, D), jnp.float32),
        mesh=_MESH,
        scratch_types=[
            pltpu.VMEM((C,), jnp.int32),
            pltpu.VMEM((C, D), jnp.float32),
            pltpu.VMEM_SHARED((N, D), jnp.float32),
        ],
    )
    def k(dst_hbm, zeros_hbm, ones_hbm, out_hbm, dst_v, ones_v, acc):
        cid = lax.axis_index("c")
        sid = lax.axis_index("s")
        wid = cid * NS + sid

        pltpu.sync_copy(ones_hbm, ones_v)
        r0 = pl.multiple_of(sid * rmain, 8)
        pltpu.sync_copy(zeros_hbm.at[pl.ds(r0, rmain)],
                        acc.at[pl.ds(r0, rmain)])

        @pl.when(sid == 0)
        def _():
            pltpu.sync_copy(zeros_hbm.at[pl.ds(tbase, tail)],
                            acc.at[pl.ds(tbase, tail)])

        plsc.subcore_barrier()

        def body(j, carry):
            base = pl.multiple_of(wid * per_w + j * C, 8)
            pltpu.sync_copy(dst_hbm.at[pl.ds(base, C)], dst_v)
            pltpu.sync_copy(ones_v, acc.at[dst_v], add=True)
            return carry

        lax.fori_loop(0, n_ch, body, 0)
        plsc.subcore_barrier()
        pltpu.sync_copy(acc.at[pl.ds(r0, rmain)],
                        out_hbm.at[cid, pl.ds(r0, rmain)])

        @pl.when(sid == 0)
        def _():
            pltpu.sync_copy(acc.at[pl.ds(tbase, tail)],
                            out_hbm.at[cid, pl.ds(tbase, tail)])

    return k


@functools.lru_cache(maxsize=None)
def _scatter_kernel(E: int, N: int, D: int):
    per_w = E // NW
    C = 80
    n_ch = per_w // C
    rmain = (N // NS) // 8 * 8
    tbase = rmain * NS
    tail = N - tbase

    @functools.partial(
        pl.kernel,
        out_type=jax.ShapeDtypeStruct((NC, N, D), jnp.float32),
        mesh=_MESH,
        scratch_types=[
            pltpu.VMEM((C,), jnp.int32),
            pltpu.VMEM((C,), jnp.int32),
            pltpu.VMEM((C, D), jnp.float32),
            pltpu.VMEM_SHARED((N, D), jnp.float32),
            pltpu.SemaphoreType.DMA,
        ],
    )
    def k(y_hbm, src_hbm, dst_hbm, out_hbm, src_v, dst_v, rows_v, acc, sem):
        cid = lax.axis_index("c")
        sid = lax.axis_index("s")
        wid = cid * NS + sid

        r0 = pl.multiple_of(sid * rmain, 8)
        pltpu.sync_copy(y_hbm.at[pl.ds(r0, rmain)], acc.at[pl.ds(r0, rmain)])

        @pl.when(sid == 0)
        def _():
            pltpu.sync_copy(y_hbm.at[pl.ds(tbase, tail)],
                            acc.at[pl.ds(tbase, tail)])

        plsc.subcore_barrier()

        def body(j, carry):
            base = pl.multiple_of(wid * per_w + j * C, 8)
            pltpu.sync_copy(src_hbm.at[pl.ds(base, C)], src_v)
            pltpu.sync_copy(dst_hbm.at[pl.ds(base, C)], dst_v)
            pltpu.async_copy(y_hbm.at[src_v], rows_v, sem).wait()
            pltpu.sync_copy(rows_v, acc.at[dst_v], add=True)
            return carry

        lax.fori_loop(0, n_ch, body, 0)
        plsc.subcore_barrier()
        pltpu.sync_copy(acc.at[pl.ds(r0, rmain)],
                        out_hbm.at[cid, pl.ds(r0, rmain)])

        @pl.when(sid == 0)
        def _():
            pltpu.sync_copy(acc.at[pl.ds(tbase, tail)],
                            out_hbm.at[cid, pl.ds(tbase, tail)])

    return k


@functools.lru_cache(maxsize=None)
def _gather_kernel(EL: int, N: int, D: int):
    per_w = EL // NW
    C = 128
    n_ch = per_w // C

    @functools.partial(
        pl.kernel,
        out_type=(jax.ShapeDtypeStruct((EL, D), jnp.float32),
                  jax.ShapeDtypeStruct((EL, D), jnp.float32)),
        mesh=_MESH,
        scratch_types=[
            pltpu.VMEM((C,), jnp.int32),
            pltpu.VMEM((C, D), jnp.float32),
            pltpu.SemaphoreType.DMA,
        ],
    )
    def k(z_hbm, a_hbm, b_hbm, zs_hbm, zd_hbm, idx_v, rows_v, sem):
        cid = lax.axis_index("c")
        sid = lax.axis_index("s")
        wid = cid * NS + sid

        def body(j, carry):
            base = pl.multiple_of(wid * per_w + j * C, 8)
            pltpu.sync_copy(a_hbm.at[pl.ds(base, C)], idx_v)
            pltpu.async_copy(z_hbm.at[idx_v], rows_v, sem).wait()
            pltpu.sync_copy(rows_v, zs_hbm.at[pl.ds(base, C)])
            pltpu.sync_copy(b_hbm.at[pl.ds(base, C)], idx_v)
            pltpu.async_copy(z_hbm.at[idx_v], rows_v, sem).wait()
            pltpu.sync_copy(rows_v, zd_hbm.at[pl.ds(base, C)])
            return carry

        lax.fori_loop(0, n_ch, body, 0)

    return k


_R = 1000


def _mm1_body(x_ref, w_ref, dp_ref, y_ref, dv_ref):
    dp = dp_ref[...]
    deg = dp[0][:, 0:1] + dp[1][:, 0:1] + 1.0
    dinv = lax.rsqrt(deg)
    y = jnp.dot(x_ref[...], w_ref[...], preferred_element_type=jnp.float32)
    y_ref[...] = y * dinv
    dv_ref[...] = jnp.broadcast_to(dinv, dv_ref.shape)


def _mm1(x, W1, degp):
    N, DI = x.shape
    DH = W1.shape[1]
    DG = degp.shape[2]
    grid = N // _R
    return pl.pallas_call(
        _mm1_body,
        grid=(grid,),
        in_specs=[
            pl.BlockSpec((_R, DI), lambda i: (i, 0)),
            pl.BlockSpec((DI, DH), lambda i: (0, 0)),
            pl.BlockSpec((NC, _R, DG), lambda i: (0, i, 0)),
        ],
        out_specs=[
            pl.BlockSpec((_R, DH), lambda i: (i, 0)),
            pl.BlockSpec((_R, LANES), lambda i: (i, 0)),
        ],
        out_shape=[
            jax.ShapeDtypeStruct((N, DH), jnp.float32),
            jax.ShapeDtypeStruct((N, LANES), jnp.float32),
        ],
    )(x, W1, degp)


def _mm2_body(sp_ref, y1_ref, dv_ref, b1_ref, w_ref, y2_ref):
    sp = sp_ref[...]
    y1 = y1_ref[...]
    dinv = dv_ref[...][:, 0:1]
    s = sp[0] + sp[1] - y1
    h = jnp.maximum(s * dinv + b1_ref[...], 0.0)
    y2_ref[...] = jnp.dot(h * dinv, w_ref[...],
                          preferred_element_type=jnp.float32)


def _mm2(s1p, y1, dv, b1, W2):
    N, DH = y1.shape
    DO = W2.shape[1]
    grid = N // _R
    return pl.pallas_call(
        _mm2_body,
        grid=(grid,),
        in_specs=[
            pl.BlockSpec((NC, _R, DH), lambda i: (0, i, 0)),
            pl.BlockSpec((_R, DH), lambda i: (i, 0)),
            pl.BlockSpec((_R, LANES), lambda i: (i, 0)),
            pl.BlockSpec((1, DH), lambda i: (0, 0)),
            pl.BlockSpec((DH, DO), lambda i: (0, 0)),
        ],
        out_specs=pl.BlockSpec((_R, DO), lambda i: (i, 0)),
        out_shape=jax.ShapeDtypeStruct((N, DO), jnp.float32),
    )(s1p, y1, dv, b1, W2)


def _fin_body(sp_ref, y2_ref, dv_ref, b2_ref, z_ref):
    sp = sp_ref[...]
    y2 = y2_ref[...]
    dinv = dv_ref[...][:, 0:1]
    z_ref[...] = (sp[0] + sp[1] - y2) * dinv + b2_ref[...]


def _fin(s2p, y2, dv, b2):
    N, DO = y2.shape
    grid = N // _R
    return pl.pallas_call(
        _fin_body,
        grid=(grid,),
        in_specs=[
            pl.BlockSpec((NC, _R, DO), lambda i: (0, i, 0)),
            pl.BlockSpec((_R, DO), lambda i: (i, 0)),
            pl.BlockSpec((_R, LANES), lambda i: (i, 0)),
            pl.BlockSpec((1, DO), lambda i: (0, 0)),
        ],
        out_specs=pl.BlockSpec((_R, DO), lambda i: (i, 0)),
        out_shape=jax.ShapeDtypeStruct((N, DO), jnp.float32),
    )(s2p, y2, dv, b2)


def _dot_body(zs_ref, zd_ref, o_ref):
    o_ref[...] = jnp.sum(zs_ref[...] * zd_ref[...], axis=1, keepdims=True)


def _dot(zs, zd):
    EL, DO = zs.shape
    RB = 4096
    grid = EL // RB
    return pl.pallas_call(
        _dot_body,
        grid=(grid,),
        in_specs=[
            pl.BlockSpec((RB, DO), lambda i: (i, 0)),
            pl.BlockSpec((RB, DO), lambda i: (i, 0)),
        ],
        out_specs=pl.BlockSpec((RB, 1), lambda i: (i, 0)),
        out_shape=jax.ShapeDtypeStruct((EL, 1), jnp.float32),
    )(zs, zd)


def kernel(x, edge_index, edge_label_index, W1, b1, W2, b2):
    N = x.shape[0]
    E = edge_index.shape[1]
    EL = edge_label_index.shape[1]
    DH = W1.shape[1]
    DO = W2.shape[1]

    src = edge_index[0]
    dst = edge_index[1]
    ea = edge_label_index[0]
    eb = edge_label_index[1]

    DP = 128
    W2p = jnp.pad(W2, ((0, 0), (0, DP - DO)))
    b2p = jnp.pad(b2, (0, DP - DO))

    degp = _deg_kernel(E, N, DP)(
        dst, jnp.zeros((N, DP), jnp.float32), jnp.ones((80, DP), jnp.float32))
    y1, dv = _mm1(x, W1, degp)
    s1p = _scatter_kernel(E, N, DH)(y1, src, dst)
    y2 = _mm2(s1p, y1, dv, b1.reshape(1, -1), W2p)
    s2p = _scatter_kernel(E, N, DP)(y2, src, dst)
    z = _fin(s2p, y2, dv, b2p.reshape(1, -1))
    zs, zd = _gather_kernel(EL, N, DP)(z, ea, eb)
    return _dot(zs, zd).reshape(-1)


# speedup vs baseline: 19.2660x; 1.7062x over previous
import functools

import jax
import jax.numpy as jnp
from jax import lax
from jax.experimental import pallas as pl
from jax.experimental.pallas import tpu as pltpu
from jax.experimental.pallas import tpu_sc as plsc

NC = 2
NS = 16
LANES = 16
NW = NC * NS

_MESH = plsc.VectorSubcoreMesh(core_axis_name="c", subcore_axis_name="s")


@functools.lru_cache(maxsize=None)
def _deg_kernel(E: int, N: int, D: int):
    per_w = E // NW
    C = 80
    n_ch = per_w // C
    rmain = (N // NS) // 8 * 8
    tbase = rmain * NS
    tail = N - tbase

    @functools.partial(
        pl.kernel,
        out_type=jax.ShapeDtypeStruct((NC, N, D), jnp.float32),
        mesh=_MESH,
        scratch_types=[
            pltpu.VMEM((2, 2, C), jnp.int32),
            pltpu.VMEM((C, D), jnp.float32),
            pltpu.VMEM_SHARED((N, D), jnp.float32),
            pltpu.SemaphoreType.DMA,
            pltpu.SemaphoreType.DMA,
        ],
    )
    def k(ec_hbm, zeros_hbm, ones_hbm, out_hbm, idx_v, ones_v, acc,
          sem0, sem1):
        cid = lax.axis_index("c")
        sid = lax.axis_index("s")
        wid = cid * NS + sid

        pltpu.sync_copy(ones_hbm, ones_v)
        r0 = pl.multiple_of(sid * rmain, 8)
        pltpu.sync_copy(zeros_hbm.at[pl.ds(r0, rmain)],
                        acc.at[pl.ds(r0, rmain)])

        @pl.when(sid == 0)
        def _():
            pltpu.sync_copy(zeros_hbm.at[pl.ds(tbase, tail)],
                            acc.at[pl.ds(tbase, tail)])

        plsc.subcore_barrier()

        sems = (sem0, sem1)

        pltpu.async_copy(ec_hbm.at[wid * n_ch], idx_v.at[0], sem0)

        @pl.loop(0, n_ch, step=2)
        def _(j):
            for b in (0, 1):
                jj = j + b
                nb = 1 - b

                @pl.when(jj + 1 < n_ch)
                def _():
                    pltpu.async_copy(ec_hbm.at[wid * n_ch + jj + 1],
                                     idx_v.at[nb], sems[nb])

                @pl.when(jj < n_ch)
                def _():
                    pltpu.make_async_copy(ec_hbm.at[wid * n_ch + jj],
                                          idx_v.at[b], sems[b]).wait()
                    pltpu.sync_copy(ones_v, acc.at[idx_v.at[b, 1]],
                                    add=True)

        plsc.subcore_barrier()
        pltpu.sync_copy(acc.at[pl.ds(r0, rmain)],
                        out_hbm.at[cid, pl.ds(r0, rmain)])

        @pl.when(sid == 0)
        def _():
            pltpu.sync_copy(acc.at[pl.ds(tbase, tail)],
                            out_hbm.at[cid, pl.ds(tbase, tail)])

    return k


@functools.lru_cache(maxsize=None)
def _scatter_kernel(E: int, N: int, D: int):
    per_w = E // NW
    C = 80
    n_ch = per_w // C
    rmain = (N // NS) // 8 * 8
    tbase = rmain * NS
    tail = N - tbase

    @functools.partial(
        pl.kernel,
        out_type=jax.ShapeDtypeStruct((NC, N, D), jnp.float32),
        mesh=_MESH,
        scratch_types=[
            pltpu.VMEM((2, 2, C), jnp.int32),
            pltpu.VMEM((2, C, D), jnp.float32),
            pltpu.VMEM_SHARED((N, D), jnp.float32),
            pltpu.SemaphoreType.DMA,
            pltpu.SemaphoreType.DMA,
        ],
    )
    def k(y_hbm, ec_hbm, out_hbm, idx_v, rows_v, acc, sem0, sem1):
        cid = lax.axis_index("c")
        sid = lax.axis_index("s")
        wid = cid * NS + sid

        r0 = pl.multiple_of(sid * rmain, 8)
        pltpu.sync_copy(y_hbm.at[pl.ds(r0, rmain)], acc.at[pl.ds(r0, rmain)])

        @pl.when(sid == 0)
        def _():
            pltpu.sync_copy(y_hbm.at[pl.ds(tbase, tail)],
                            acc.at[pl.ds(tbase, tail)])

        plsc.subcore_barrier()

        sems = (sem0, sem1)

        def stage(jj, b):
            pltpu.sync_copy(ec_hbm.at[wid * n_ch + jj], idx_v.at[b])
            pltpu.async_copy(y_hbm.at[idx_v.at[b, 0]], rows_v.at[b],
                             sems[b])

        stage(0, 0)

        @pl.loop(0, n_ch, step=2)
        def _(j):
            for b in (0, 1):
                jj = j + b
                nb = 1 - b
                @pl.when(jj + 1 < n_ch)
                def _():
                    stage(jj + 1, nb)

                @pl.when(jj < n_ch)
                def _():
                    pltpu.make_async_copy(y_hbm.at[idx_v.at[b, 0]],
                                          rows_v.at[b], sems[b]).wait()
                    pltpu.sync_copy(rows_v.at[b], acc.at[idx_v.at[b, 1]],
                                    add=True)

        plsc.subcore_barrier()
        pltpu.sync_copy(acc.at[pl.ds(r0, rmain)],
                        out_hbm.at[cid, pl.ds(r0, rmain)])

        @pl.when(sid == 0)
        def _():
            pltpu.sync_copy(acc.at[pl.ds(tbase, tail)],
                            out_hbm.at[cid, pl.ds(tbase, tail)])

    return k


@functools.lru_cache(maxsize=None)
def _gather_kernel(EL: int, N: int, D: int):
    per_w = EL // NW
    C = 64
    n_ch = per_w // C

    @functools.partial(
        pl.kernel,
        out_type=(jax.ShapeDtypeStruct((EL, D), jnp.float32),
                  jax.ShapeDtypeStruct((EL, D), jnp.float32)),
        mesh=_MESH,
        scratch_types=[
            pltpu.VMEM((2, 2, C), jnp.int32),
            pltpu.VMEM((2, C, D), jnp.float32),
            pltpu.VMEM((2, C, D), jnp.float32),
            pltpu.SemaphoreType.DMA,
            pltpu.SemaphoreType.DMA,
            pltpu.SemaphoreType.DMA,
            pltpu.SemaphoreType.DMA,
        ],
    )
    def k(z_hbm, ec_hbm, zs_hbm, zd_hbm, idx_v, ra_v, rb_v,
          sa0, sa1, sb0, sb1):
        cid = lax.axis_index("c")
        sid = lax.axis_index("s")
        wid = cid * NS + sid

        sas = (sa0, sa1)
        sbs = (sb0, sb1)

        def idx_of(j):
            return pl.ds(pl.multiple_of(wid * per_w + j * C, 8), C)

        def stage(jj, b):
            pltpu.sync_copy(ec_hbm.at[wid * n_ch + jj], idx_v.at[b])
            pltpu.async_copy(z_hbm.at[idx_v.at[b, 0]], ra_v.at[b], sas[b])
            pltpu.async_copy(z_hbm.at[idx_v.at[b, 1]], rb_v.at[b], sbs[b])

        stage(0, 0)

        @pl.loop(0, n_ch, step=2)
        def _(j):
            for b in (0, 1):
                jj = j + b
                nb = 1 - b

                @pl.when(jj + 1 < n_ch)
                def _():
                    stage(jj + 1, nb)

                @pl.when(jj < n_ch)
                def _():
                    pltpu.make_async_copy(z_hbm.at[idx_v.at[b, 0]],
                                          ra_v.at[b], sas[b]).wait()
                    pltpu.sync_copy(ra_v.at[b], zs_hbm.at[idx_of(jj)])
                    pltpu.make_async_copy(z_hbm.at[idx_v.at[b, 1]],
                                          rb_v.at[b], sbs[b]).wait()
                    pltpu.sync_copy(rb_v.at[b], zd_hbm.at[idx_of(jj)])

    return k


_R = 1000


def _mm1_body(x_ref, w_ref, dp_ref, y_ref, dv_ref):
    dp = dp_ref[...]
    deg = dp[0][:, 0:1] + dp[1][:, 0:1] + 1.0
    dinv = lax.rsqrt(deg)
    y = jnp.dot(x_ref[...], w_ref[...], preferred_element_type=jnp.float32)
    y_ref[...] = y * dinv
    dv_ref[...] = jnp.broadcast_to(dinv, dv_ref.shape)


def _mm1(x, W1, degp):
    N, DI = x.shape
    DH = W1.shape[1]
    DG = degp.shape[2]
    grid = N // _R
    return pl.pallas_call(
        _mm1_body,
        grid=(grid,),
        in_specs=[
            pl.BlockSpec((_R, DI), lambda i: (i, 0)),
            pl.BlockSpec((DI, DH), lambda i: (0, 0)),
            pl.BlockSpec((NC, _R, DG), lambda i: (0, i, 0)),
        ],
        out_specs=[
            pl.BlockSpec((_R, DH), lambda i: (i, 0)),
            pl.BlockSpec((_R, LANES), lambda i: (i, 0)),
        ],
        out_shape=[
            jax.ShapeDtypeStruct((N, DH), jnp.float32),
            jax.ShapeDtypeStruct((N, LANES), jnp.float32),
        ],
    )(x, W1, degp)


def _mm2_body(sp_ref, y1_ref, dv_ref, b1_ref, w_ref, y2_ref):
    sp = sp_ref[...]
    y1 = y1_ref[...]
    dinv = dv_ref[...][:, 0:1]
    s = sp[0] + sp[1] - y1
    h = jnp.maximum(s * dinv + b1_ref[...], 0.0)
    y2_ref[...] = jnp.dot(h * dinv, w_ref[...],
                          preferred_element_type=jnp.float32)


def _mm2(s1p, y1, dv, b1, W2):
    N, DH = y1.shape
    DO = W2.shape[1]
    grid = N // _R
    return pl.pallas_call(
        _mm2_body,
        grid=(grid,),
        in_specs=[
            pl.BlockSpec((NC, _R, DH), lambda i: (0, i, 0)),
            pl.BlockSpec((_R, DH), lambda i: (i, 0)),
            pl.BlockSpec((_R, LANES), lambda i: (i, 0)),
            pl.BlockSpec((1, DH), lambda i: (0, 0)),
            pl.BlockSpec((DH, DO), lambda i: (0, 0)),
        ],
        out_specs=pl.BlockSpec((_R, DO), lambda i: (i, 0)),
        out_shape=jax.ShapeDtypeStruct((N, DO), jnp.float32),
    )(s1p, y1, dv, b1, W2)


def _fin_body(sp_ref, y2_ref, dv_ref, b2_ref, z_ref):
    sp = sp_ref[...]
    y2 = y2_ref[...]
    dinv = dv_ref[...][:, 0:1]
    z_ref[...] = (sp[0] + sp[1] - y2) * dinv + b2_ref[...]


def _fin(s2p, y2, dv, b2):
    N, DO = y2.shape
    grid = N // _R
    return pl.pallas_call(
        _fin_body,
        grid=(grid,),
        in_specs=[
            pl.BlockSpec((NC, _R, DO), lambda i: (0, i, 0)),
            pl.BlockSpec((_R, DO), lambda i: (i, 0)),
            pl.BlockSpec((_R, LANES), lambda i: (i, 0)),
            pl.BlockSpec((1, DO), lambda i: (0, 0)),
        ],
        out_specs=pl.BlockSpec((_R, DO), lambda i: (i, 0)),
        out_shape=jax.ShapeDtypeStruct((N, DO), jnp.float32),
    )(s2p, y2, dv, b2)


def _dot_body(zs_ref, zd_ref, o_ref):
    o_ref[...] = jnp.sum(zs_ref[...] * zd_ref[...], axis=1, keepdims=True)


def _dot(zs, zd):
    EL, DO = zs.shape
    RB = 4096
    grid = EL // RB
    return pl.pallas_call(
        _dot_body,
        grid=(grid,),
        in_specs=[
            pl.BlockSpec((RB, DO), lambda i: (i, 0)),
            pl.BlockSpec((RB, DO), lambda i: (i, 0)),
        ],
        out_specs=pl.BlockSpec((RB, 1), lambda i: (i, 0)),
        out_shape=jax.ShapeDtypeStruct((EL, 1), jnp.float32),
    )(zs, zd)


def kernel(x, edge_index, edge_label_index, W1, b1, W2, b2):
    N = x.shape[0]
    E = edge_index.shape[1]
    EL = edge_label_index.shape[1]
    DH = W1.shape[1]
    DO = W2.shape[1]

    DP = 128
    W2p = jnp.pad(W2, ((0, 0), (0, DP - DO)))
    b2p = jnp.pad(b2, (0, DP - DO))

    ec = jnp.transpose(edge_index.reshape(2, E // 80, 80), (1, 0, 2))
    elc = jnp.transpose(edge_label_index.reshape(2, EL // 64, 64), (1, 0, 2))

    degp = _deg_kernel(E, N, DP)(
        ec, jnp.zeros((N, DP), jnp.float32), jnp.ones((80, DP), jnp.float32))
    y1, dv = _mm1(x, W1, degp)
    s1p = _scatter_kernel(E, N, DH)(y1, ec)
    y2 = _mm2(s1p, y1, dv, b1.reshape(1, -1), W2p)
    s2p = _scatter_kernel(E, N, DP)(y2, ec)
    z = _fin(s2p, y2, dv, b2p.reshape(1, -1))
    zs, zd = _gather_kernel(EL, N, DP)(z, elc)
    return _dot(zs, zd).reshape(-1)


# speedup vs baseline: 19.2816x; 1.0008x over previous
import functools

import jax
import jax.numpy as jnp
from jax import lax
from jax.experimental import pallas as pl
from jax.experimental.pallas import tpu as pltpu
from jax.experimental.pallas import tpu_sc as plsc

NC = 2
NS = 16
LANES = 16
NW = NC * NS

_MESH = plsc.VectorSubcoreMesh(core_axis_name="c", subcore_axis_name="s")


@functools.lru_cache(maxsize=None)
def _deg_kernel(E: int, N: int, D: int):
    per_w = E // NW
    C = 80
    n_ch = per_w // C
    rmain = (N // NS) // 8 * 8
    tbase = rmain * NS
    tail = N - tbase

    @functools.partial(
        pl.kernel,
        out_type=jax.ShapeDtypeStruct((NC, N, D), jnp.float32),
        mesh=_MESH,
        scratch_types=[
            pltpu.VMEM((2, 2, C), jnp.int32),
            pltpu.VMEM((C, D), jnp.float32),
            pltpu.VMEM_SHARED((N, D), jnp.float32),
            pltpu.SemaphoreType.DMA,
            pltpu.SemaphoreType.DMA,
            pltpu.SemaphoreType.DMA,
            pltpu.SemaphoreType.DMA,
        ],
    )
    def k(ec_hbm, zeros_hbm, ones_hbm, out_hbm, idx_v, ones_v, acc,
          sem0, sem1, ssem0, ssem1):
        cid = lax.axis_index("c")
        sid = lax.axis_index("s")
        wid = cid * NS + sid

        pltpu.sync_copy(ones_hbm, ones_v)
        r0 = pl.multiple_of(sid * rmain, 8)
        pltpu.sync_copy(zeros_hbm.at[pl.ds(r0, rmain)],
                        acc.at[pl.ds(r0, rmain)])

        @pl.when(sid == 0)
        def _():
            pltpu.sync_copy(zeros_hbm.at[pl.ds(tbase, tail)],
                            acc.at[pl.ds(tbase, tail)])

        plsc.subcore_barrier()

        sems = (sem0, sem1)
        ssems = (ssem0, ssem1)

        def scat_desc(b):
            return pltpu.make_async_copy(ones_v, acc.at[idx_v.at[b, 1]],
                                         ssems[b])

        pltpu.async_copy(ec_hbm.at[wid * n_ch], idx_v.at[0], sem0)

        @pl.loop(0, n_ch, step=2)
        def _(j):
            for b in (0, 1):
                jj = j + b
                nb = 1 - b

                @pl.when(jj + 1 < n_ch)
                def _():
                    @pl.when(jj >= 1)
                    def _():
                        scat_desc(nb).wait()

                    pltpu.async_copy(ec_hbm.at[wid * n_ch + jj + 1],
                                     idx_v.at[nb], sems[nb])

                @pl.when(jj < n_ch)
                def _():
                    pltpu.make_async_copy(ec_hbm.at[wid * n_ch + jj],
                                          idx_v.at[b], sems[b]).wait()
                    pltpu.async_copy(ones_v, acc.at[idx_v.at[b, 1]],
                                     ssems[b], add=True)

        scat_desc((n_ch - 2) % 2).wait()
        scat_desc((n_ch - 1) % 2).wait()
        plsc.subcore_barrier()
        pltpu.sync_copy(acc.at[pl.ds(r0, rmain)],
                        out_hbm.at[cid, pl.ds(r0, rmain)])

        @pl.when(sid == 0)
        def _():
            pltpu.sync_copy(acc.at[pl.ds(tbase, tail)],
                            out_hbm.at[cid, pl.ds(tbase, tail)])

    return k


@functools.lru_cache(maxsize=None)
def _scatter_kernel(E: int, N: int, D: int):
    per_w = E // NW
    C = 80
    n_ch = per_w // C
    rmain = (N // NS) // 8 * 8
    tbase = rmain * NS
    tail = N - tbase

    @functools.partial(
        pl.kernel,
        out_type=jax.ShapeDtypeStruct((NC, N, D), jnp.float32),
        mesh=_MESH,
        scratch_types=[
            pltpu.VMEM((2, 2, C), jnp.int32),
            pltpu.VMEM((2, C, D), jnp.float32),
            pltpu.VMEM_SHARED((N, D), jnp.float32),
            pltpu.SemaphoreType.DMA,
            pltpu.SemaphoreType.DMA,
            pltpu.SemaphoreType.DMA,
            pltpu.SemaphoreType.DMA,
        ],
    )
    def k(y_hbm, ec_hbm, out_hbm, idx_v, rows_v, acc,
          gsem0, gsem1, ssem0, ssem1):
        cid = lax.axis_index("c")
        sid = lax.axis_index("s")
        wid = cid * NS + sid

        r0 = pl.multiple_of(sid * rmain, 8)
        pltpu.sync_copy(y_hbm.at[pl.ds(r0, rmain)], acc.at[pl.ds(r0, rmain)])

        @pl.when(sid == 0)
        def _():
            pltpu.sync_copy(y_hbm.at[pl.ds(tbase, tail)],
                            acc.at[pl.ds(tbase, tail)])

        plsc.subcore_barrier()

        gsems = (gsem0, gsem1)
        ssems = (ssem0, ssem1)

        def stage(jj, b):
            pltpu.sync_copy(ec_hbm.at[wid * n_ch + jj], idx_v.at[b])
            pltpu.async_copy(y_hbm.at[idx_v.at[b, 0]], rows_v.at[b],
                             gsems[b])

        def scat_desc(b):
            return pltpu.make_async_copy(rows_v.at[b],
                                         acc.at[idx_v.at[b, 1]], ssems[b])

        stage(0, 0)

        @pl.loop(0, n_ch, step=2)
        def _(j):
            for b in (0, 1):
                jj = j + b
                nb = 1 - b

                @pl.when(jj + 1 < n_ch)
                def _():
                    @pl.when(jj >= 1)
                    def _():
                        scat_desc(nb).wait()

                    stage(jj + 1, nb)

                @pl.when(jj < n_ch)
                def _():
                    pltpu.make_async_copy(y_hbm.at[idx_v.at[b, 0]],
                                          rows_v.at[b], gsems[b]).wait()
                    pltpu.async_copy(rows_v.at[b], acc.at[idx_v.at[b, 1]],
                                     ssems[b], add=True)

        scat_desc((n_ch - 2) % 2).wait()
        scat_desc((n_ch - 1) % 2).wait()
        plsc.subcore_barrier()
        pltpu.sync_copy(acc.at[pl.ds(r0, rmain)],
                        out_hbm.at[cid, pl.ds(r0, rmain)])

        @pl.when(sid == 0)
        def _():
            pltpu.sync_copy(acc.at[pl.ds(tbase, tail)],
                            out_hbm.at[cid, pl.ds(tbase, tail)])

    return k


@functools.lru_cache(maxsize=None)
def _gather_kernel(EL: int, N: int, D: int):
    per_w = EL // NW
    C = 64
    n_ch = per_w // C

    @functools.partial(
        pl.kernel,
        out_type=(jax.ShapeDtypeStruct((EL, D), jnp.float32),
                  jax.ShapeDtypeStruct((EL, D), jnp.float32)),
        mesh=_MESH,
        scratch_types=[
            pltpu.VMEM((2, 2, C), jnp.int32),
            pltpu.VMEM((2, C, D), jnp.float32),
            pltpu.VMEM((2, C, D), jnp.float32),
            pltpu.SemaphoreType.DMA,
            pltpu.SemaphoreType.DMA,
            pltpu.SemaphoreType.DMA,
            pltpu.SemaphoreType.DMA,
        ],
    )
    def k(z_hbm, ec_hbm, zs_hbm, zd_hbm, idx_v, ra_v, rb_v,
          sa0, sa1, sb0, sb1):
        cid = lax.axis_index("c")
        sid = lax.axis_index("s")
        wid = cid * NS + sid

        sas = (sa0, sa1)
        sbs = (sb0, sb1)

        def idx_of(j):
            return pl.ds(pl.multiple_of(wid * per_w + j * C, 8), C)

        def stage(jj, b):
            pltpu.sync_copy(ec_hbm.at[wid * n_ch + jj], idx_v.at[b])
            pltpu.async_copy(z_hbm.at[idx_v.at[b, 0]], ra_v.at[b], sas[b])
            pltpu.async_copy(z_hbm.at[idx_v.at[b, 1]], rb_v.at[b], sbs[b])

        stage(0, 0)

        @pl.loop(0, n_ch, step=2)
        def _(j):
            for b in (0, 1):
                jj = j + b
                nb = 1 - b

                @pl.when(jj + 1 < n_ch)
                def _():
                    stage(jj + 1, nb)

                @pl.when(jj < n_ch)
                def _():
                    pltpu.make_async_copy(z_hbm.at[idx_v.at[b, 0]],
                                          ra_v.at[b], sas[b]).wait()
                    pltpu.sync_copy(ra_v.at[b], zs_hbm.at[idx_of(jj)])
                    pltpu.make_async_copy(z_hbm.at[idx_v.at[b, 1]],
                                          rb_v.at[b], sbs[b]).wait()
                    pltpu.sync_copy(rb_v.at[b], zd_hbm.at[idx_of(jj)])

    return k


_R = 1000


def _mm1_body(x_ref, w_ref, dp_ref, y_ref, dv_ref):
    dp = dp_ref[...]
    deg = dp[0][:, 0:1] + dp[1][:, 0:1] + 1.0
    dinv = lax.rsqrt(deg)
    y = jnp.dot(x_ref[...], w_ref[...], preferred_element_type=jnp.float32)
    y_ref[...] = y * dinv
    dv_ref[...] = jnp.broadcast_to(dinv, dv_ref.shape)


def _mm1(x, W1, degp):
    N, DI = x.shape
    DH = W1.shape[1]
    DG = degp.shape[2]
    grid = N // _R
    return pl.pallas_call(
        _mm1_body,
        grid=(grid,),
        in_specs=[
            pl.BlockSpec((_R, DI), lambda i: (i, 0)),
            pl.BlockSpec((DI, DH), lambda i: (0, 0)),
            pl.BlockSpec((NC, _R, DG), lambda i: (0, i, 0)),
        ],
        out_specs=[
            pl.BlockSpec((_R, DH), lambda i: (i, 0)),
            pl.BlockSpec((_R, LANES), lambda i: (i, 0)),
        ],
        out_shape=[
            jax.ShapeDtypeStruct((N, DH), jnp.float32),
            jax.ShapeDtypeStruct((N, LANES), jnp.float32),
        ],
    )(x, W1, degp)


def _mm2_body(sp_ref, y1_ref, dv_ref, b1_ref, w_ref, y2_ref):
    sp = sp_ref[...]
    y1 = y1_ref[...]
    dinv = dv_ref[...][:, 0:1]
    s = sp[0] + sp[1] - y1
    h = jnp.maximum(s * dinv + b1_ref[...], 0.0)
    y2_ref[...] = jnp.dot(h * dinv, w_ref[...],
                          preferred_element_type=jnp.float32)


def _mm2(s1p, y1, dv, b1, W2):
    N, DH = y1.shape
    DO = W2.shape[1]
    grid = N // _R
    return pl.pallas_call(
        _mm2_body,
        grid=(grid,),
        in_specs=[
            pl.BlockSpec((NC, _R, DH), lambda i: (0, i, 0)),
            pl.BlockSpec((_R, DH), lambda i: (i, 0)),
            pl.BlockSpec((_R, LANES), lambda i: (i, 0)),
            pl.BlockSpec((1, DH), lambda i: (0, 0)),
            pl.BlockSpec((DH, DO), lambda i: (0, 0)),
        ],
        out_specs=pl.BlockSpec((_R, DO), lambda i: (i, 0)),
        out_shape=jax.ShapeDtypeStruct((N, DO), jnp.float32),
    )(s1p, y1, dv, b1, W2)


def _fin_body(sp_ref, y2_ref, dv_ref, b2_ref, z_ref):
    sp = sp_ref[...]
    y2 = y2_ref[...]
    dinv = dv_ref[...][:, 0:1]
    z_ref[...] = (sp[0] + sp[1] - y2) * dinv + b2_ref[...]


def _fin(s2p, y2, dv, b2):
    N, DO = y2.shape
    grid = N // _R
    return pl.pallas_call(
        _fin_body,
        grid=(grid,),
        in_specs=[
            pl.BlockSpec((NC, _R, DO), lambda i: (0, i, 0)),
            pl.BlockSpec((_R, DO), lambda i: (i, 0)),
            pl.BlockSpec((_R, LANES), lambda i: (i, 0)),
            pl.BlockSpec((1, DO), lambda i: (0, 0)),
        ],
        out_specs=pl.BlockSpec((_R, DO), lambda i: (i, 0)),
        out_shape=jax.ShapeDtypeStruct((N, DO), jnp.float32),
    )(s2p, y2, dv, b2)


def _dot_body(zs_ref, zd_ref, o_ref):
    o_ref[...] = jnp.sum(zs_ref[...] * zd_ref[...], axis=1, keepdims=True)


def _dot(zs, zd):
    EL, DO = zs.shape
    RB = 4096
    grid = EL // RB
    return pl.pallas_call(
        _dot_body,
        grid=(grid,),
        in_specs=[
            pl.BlockSpec((RB, DO), lambda i: (i, 0)),
            pl.BlockSpec((RB, DO), lambda i: (i, 0)),
        ],
        out_specs=pl.BlockSpec((RB, 1), lambda i: (i, 0)),
        out_shape=jax.ShapeDtypeStruct((EL, 1), jnp.float32),
    )(zs, zd)


def kernel(x, edge_index, edge_label_index, W1, b1, W2, b2):
    N = x.shape[0]
    E = edge_index.shape[1]
    EL = edge_label_index.shape[1]
    DH = W1.shape[1]
    DO = W2.shape[1]

    DP = 128
    W2p = jnp.pad(W2, ((0, 0), (0, DP - DO)))
    b2p = jnp.pad(b2, (0, DP - DO))

    ec = jnp.transpose(edge_index.reshape(2, E // 80, 80), (1, 0, 2))
    elc = jnp.transpose(edge_label_index.reshape(2, EL // 64, 64), (1, 0, 2))

    degp = _deg_kernel(E, N, DP)(
        ec, jnp.zeros((N, DP), jnp.float32), jnp.ones((80, DP), jnp.float32))
    y1, dv = _mm1(x, W1, degp)
    s1p = _scatter_kernel(E, N, DH)(y1, ec)
    y2 = _mm2(s1p, y1, dv, b1.reshape(1, -1), W2p)
    s2p = _scatter_kernel(E, N, DP)(y2, ec)
    z = _fin(s2p, y2, dv, b2p.reshape(1, -1))
    zs, zd = _gather_kernel(EL, N, DP)(z, elc)
    return _dot(zs, zd).reshape(-1)


# speedup vs baseline: 21.8105x; 1.1312x over previous
import functools

import jax
import jax.numpy as jnp
from jax import lax
from jax.experimental import pallas as pl
from jax.experimental.pallas import tpu as pltpu
from jax.experimental.pallas import tpu_sc as plsc

NC = 2
NS = 16
LANES = 16
NW = NC * NS

_MESH = plsc.VectorSubcoreMesh(core_axis_name="c", subcore_axis_name="s")


@functools.lru_cache(maxsize=None)
def _deg_kernel(E: int, N: int, D: int):
    per_w = E // NW
    C = 80
    n_ch = per_w // C
    rmain = (N // NS) // 8 * 8
    tbase = rmain * NS
    tail = N - tbase

    @functools.partial(
        pl.kernel,
        out_type=jax.ShapeDtypeStruct((NC, N, D), jnp.float32),
        mesh=_MESH,
        scratch_types=[
            pltpu.VMEM((per_w // C, 2, C), jnp.int32),
            pltpu.VMEM((C, D), jnp.float32),
            pltpu.VMEM_SHARED((N, D), jnp.float32),
            pltpu.SemaphoreType.DMA,
            pltpu.SemaphoreType.DMA,
        ],
    )
    def k(ec_hbm, zeros_hbm, ones_hbm, out_hbm, idx_v, ones_v, acc,
          ssem0, ssem1):
        cid = lax.axis_index("c")
        sid = lax.axis_index("s")
        wid = cid * NS + sid

        pltpu.sync_copy(ones_hbm, ones_v)
        pltpu.sync_copy(ec_hbm.at[pl.ds(wid * n_ch, n_ch)], idx_v)
        r0 = pl.multiple_of(sid * rmain, 8)
        pltpu.sync_copy(zeros_hbm.at[pl.ds(r0, rmain)],
                        acc.at[pl.ds(r0, rmain)])

        @pl.when(sid == 0)
        def _():
            pltpu.sync_copy(zeros_hbm.at[pl.ds(tbase, tail)],
                            acc.at[pl.ds(tbase, tail)])

        plsc.subcore_barrier()

        ssems = (ssem0, ssem1)

        def scat_desc(jj, b):
            return pltpu.make_async_copy(ones_v, acc.at[idx_v.at[jj, 1]],
                                         ssems[b])

        @pl.loop(0, n_ch, step=2)
        def _(j):
            for b in (0, 1):
                jj = j + b

                @pl.when(jj < n_ch)
                def _():
                    @pl.when(jj >= 2)
                    def _():
                        scat_desc(jj - 2, b).wait()

                    pltpu.async_copy(ones_v, acc.at[idx_v.at[jj, 1]],
                                     ssems[b], add=True)

        scat_desc(n_ch - 2, (n_ch - 2) % 2).wait()
        scat_desc(n_ch - 1, (n_ch - 1) % 2).wait()
        plsc.subcore_barrier()
        pltpu.sync_copy(acc.at[pl.ds(r0, rmain)],
                        out_hbm.at[cid, pl.ds(r0, rmain)])

        @pl.when(sid == 0)
        def _():
            pltpu.sync_copy(acc.at[pl.ds(tbase, tail)],
                            out_hbm.at[cid, pl.ds(tbase, tail)])

    return k


@functools.lru_cache(maxsize=None)
def _scatter_kernel(E: int, N: int, D: int):
    per_w = E // NW
    C = 80
    n_ch = per_w // C
    rmain = (N // NS) // 8 * 8
    tbase = rmain * NS
    tail = N - tbase

    half = (n_ch + 1) // 2
    phases = ((0, half), (half, n_ch - half))

    @functools.partial(
        pl.kernel,
        out_type=jax.ShapeDtypeStruct((NC, N, D), jnp.float32),
        mesh=_MESH,
        scratch_types=[
            pltpu.VMEM((half, 2, C), jnp.int32),
            pltpu.VMEM((2, C, D), jnp.float32),
            pltpu.VMEM_SHARED((N, D), jnp.float32),
            pltpu.SemaphoreType.DMA,
            pltpu.SemaphoreType.DMA,
            pltpu.SemaphoreType.DMA,
            pltpu.SemaphoreType.DMA,
        ],
    )
    def k(y_hbm, ec_hbm, out_hbm, idx_v, rows_v, acc,
          gsem0, gsem1, ssem0, ssem1):
        cid = lax.axis_index("c")
        sid = lax.axis_index("s")
        wid = cid * NS + sid

        pltpu.sync_copy(ec_hbm.at[pl.ds(wid * n_ch, half)],
                        idx_v.at[pl.ds(0, half)])
        r0 = pl.multiple_of(sid * rmain, 8)
        pltpu.sync_copy(y_hbm.at[pl.ds(r0, rmain)], acc.at[pl.ds(r0, rmain)])

        @pl.when(sid == 0)
        def _():
            pltpu.sync_copy(y_hbm.at[pl.ds(tbase, tail)],
                            acc.at[pl.ds(tbase, tail)])

        plsc.subcore_barrier()

        gsems = (gsem0, gsem1)
        ssems = (ssem0, ssem1)

        def gath_desc(jj, b):
            return pltpu.make_async_copy(y_hbm.at[idx_v.at[jj, 0]],
                                         rows_v.at[b], gsems[b])

        def scat_desc(jj, b):
            return pltpu.make_async_copy(rows_v.at[b],
                                         acc.at[idx_v.at[jj, 1]], ssems[b])

        for c0, cn in phases:
            if c0 > 0:
                pltpu.sync_copy(ec_hbm.at[pl.ds(wid * n_ch + c0, cn)],
                                idx_v.at[pl.ds(0, cn)])
            pltpu.async_copy(y_hbm.at[idx_v.at[0, 0]], rows_v.at[0], gsem0)

            @pl.loop(0, cn, step=2)
            def _(j):
                for b in (0, 1):
                    jj = j + b
                    nb = 1 - b

                    @pl.when(jj + 1 < cn)
                    def _():
                        @pl.when(jj >= 1)
                        def _():
                            scat_desc(jj - 1, nb).wait()

                        pltpu.async_copy(y_hbm.at[idx_v.at[jj + 1, 0]],
                                         rows_v.at[nb], gsems[nb])

                    @pl.when(jj < cn)
                    def _():
                        gath_desc(jj, b).wait()
                        pltpu.async_copy(rows_v.at[b],
                                         acc.at[idx_v.at[jj, 1]],
                                         ssems[b], add=True)

            scat_desc(cn - 2, (cn - 2) % 2).wait()
            scat_desc(cn - 1, (cn - 1) % 2).wait()

        plsc.subcore_barrier()
        pltpu.sync_copy(acc.at[pl.ds(r0, rmain)],
                        out_hbm.at[cid, pl.ds(r0, rmain)])

        @pl.when(sid == 0)
        def _():
            pltpu.sync_copy(acc.at[pl.ds(tbase, tail)],
                            out_hbm.at[cid, pl.ds(tbase, tail)])

    return k


@functools.lru_cache(maxsize=None)
def _gather_kernel(EL: int, N: int, D: int):
    per_w = EL // NW
    C = 64
    n_ch = per_w // C

    @functools.partial(
        pl.kernel,
        out_type=(jax.ShapeDtypeStruct((EL, D), jnp.float32),
                  jax.ShapeDtypeStruct((EL, D), jnp.float32)),
        mesh=_MESH,
        scratch_types=[
            pltpu.VMEM((per_w // C, 2, C), jnp.int32),
            pltpu.VMEM((2, C, D), jnp.float32),
            pltpu.VMEM((2, C, D), jnp.float32),
            pltpu.SemaphoreType.DMA,
            pltpu.SemaphoreType.DMA,
            pltpu.SemaphoreType.DMA,
            pltpu.SemaphoreType.DMA,
            pltpu.SemaphoreType.DMA,
            pltpu.SemaphoreType.DMA,
            pltpu.SemaphoreType.DMA,
            pltpu.SemaphoreType.DMA,
        ],
    )
    def k(z_hbm, ec_hbm, zs_hbm, zd_hbm, idx_v, ra_v, rb_v,
          sa0, sa1, sb0, sb1, oa0, oa1, ob0, ob1):
        cid = lax.axis_index("c")
        sid = lax.axis_index("s")
        wid = cid * NS + sid

        sas = (sa0, sa1)
        sbs = (sb0, sb1)
        oas = (oa0, oa1)
        obs = (ob0, ob1)

        pltpu.sync_copy(ec_hbm.at[pl.ds(wid * n_ch, n_ch)], idx_v)

        def out_of(j):
            return pl.ds(pl.multiple_of(wid * per_w + j * C, 8), C)

        def gathers(jj, b):
            pltpu.async_copy(z_hbm.at[idx_v.at[jj, 0]], ra_v.at[b], sas[b])
            pltpu.async_copy(z_hbm.at[idx_v.at[jj, 1]], rb_v.at[b], sbs[b])

        def odesc(jj, b):
            return (pltpu.make_async_copy(ra_v.at[b], zs_hbm.at[out_of(jj)],
                                          oas[b]),
                    pltpu.make_async_copy(rb_v.at[b], zd_hbm.at[out_of(jj)],
                                          obs[b]))

        gathers(0, 0)

        @pl.loop(0, n_ch, step=2)
        def _(j):
            for b in (0, 1):
                jj = j + b
                nb = 1 - b

                @pl.when(jj + 1 < n_ch)
                def _():
                    @pl.when(jj >= 1)
                    def _():
                        da, db = odesc(jj - 1, nb)
                        da.wait()
                        db.wait()

                    gathers(jj + 1, nb)

                @pl.when(jj < n_ch)
                def _():
                    pltpu.make_async_copy(z_hbm.at[idx_v.at[jj, 0]],
                                          ra_v.at[b], sas[b]).wait()
                    pltpu.async_copy(ra_v.at[b], zs_hbm.at[out_of(jj)],
                                     oas[b])
                    pltpu.make_async_copy(z_hbm.at[idx_v.at[jj, 1]],
                                          rb_v.at[b], sbs[b]).wait()
                    pltpu.async_copy(rb_v.at[b], zd_hbm.at[out_of(jj)],
                                     obs[b])

        for last in (n_ch - 2, n_ch - 1):
            da, db = odesc(last, last % 2)
            da.wait()
            db.wait()

    return k


_R = 1000


def _mm1_body(x_ref, w_ref, dp_ref, y_ref, dv_ref):
    dp = dp_ref[...]
    deg = dp[0][:, 0:1] + dp[1][:, 0:1] + 1.0
    dinv = lax.rsqrt(deg)
    y = jnp.dot(x_ref[...], w_ref[...], preferred_element_type=jnp.float32)
    y_ref[...] = y * dinv
    dv_ref[...] = jnp.broadcast_to(dinv, dv_ref.shape)


def _mm1(x, W1, degp):
    N, DI = x.shape
    DH = W1.shape[1]
    DG = degp.shape[2]
    grid = N // _R
    return pl.pallas_call(
        _mm1_body,
        grid=(grid,),
        in_specs=[
            pl.BlockSpec((_R, DI), lambda i: (i, 0)),
            pl.BlockSpec((DI, DH), lambda i: (0, 0)),
            pl.BlockSpec((NC, _R, DG), lambda i: (0, i, 0)),
        ],
        out_specs=[
            pl.BlockSpec((_R, DH), lambda i: (i, 0)),
            pl.BlockSpec((_R, LANES), lambda i: (i, 0)),
        ],
        out_shape=[
            jax.ShapeDtypeStruct((N, DH), jnp.float32),
            jax.ShapeDtypeStruct((N, LANES), jnp.float32),
        ],
    )(x, W1, degp)


def _mm2_body(sp_ref, y1_ref, dv_ref, b1_ref, w_ref, y2_ref):
    sp = sp_ref[...]
    y1 = y1_ref[...]
    dinv = dv_ref[...][:, 0:1]
    s = sp[0] + sp[1] - y1
    h = jnp.maximum(s * dinv + b1_ref[...], 0.0)
    y2_ref[...] = jnp.dot(h * dinv, w_ref[...],
                          preferred_element_type=jnp.float32)


def _mm2(s1p, y1, dv, b1, W2):
    N, DH = y1.shape
    DO = W2.shape[1]
    grid = N // _R
    return pl.pallas_call(
        _mm2_body,
        grid=(grid,),
        in_specs=[
            pl.BlockSpec((NC, _R, DH), lambda i: (0, i, 0)),
            pl.BlockSpec((_R, DH), lambda i: (i, 0)),
            pl.BlockSpec((_R, LANES), lambda i: (i, 0)),
            pl.BlockSpec((1, DH), lambda i: (0, 0)),
            pl.BlockSpec((DH, DO), lambda i: (0, 0)),
        ],
        out_specs=pl.BlockSpec((_R, DO), lambda i: (i, 0)),
        out_shape=jax.ShapeDtypeStruct((N, DO), jnp.float32),
    )(s1p, y1, dv, b1, W2)


def _fin_body(sp_ref, y2_ref, dv_ref, b2_ref, z_ref):
    sp = sp_ref[...]
    y2 = y2_ref[...]
    dinv = dv_ref[...][:, 0:1]
    z_ref[...] = (sp[0] + sp[1] - y2) * dinv + b2_ref[...]


def _fin(s2p, y2, dv, b2):
    N, DO = y2.shape
    grid = N // _R
    return pl.pallas_call(
        _fin_body,
        grid=(grid,),
        in_specs=[
            pl.BlockSpec((NC, _R, DO), lambda i: (0, i, 0)),
            pl.BlockSpec((_R, DO), lambda i: (i, 0)),
            pl.BlockSpec((_R, LANES), lambda i: (i, 0)),
            pl.BlockSpec((1, DO), lambda i: (0, 0)),
        ],
        out_specs=pl.BlockSpec((_R, DO), lambda i: (i, 0)),
        out_shape=jax.ShapeDtypeStruct((N, DO), jnp.float32),
    )(s2p, y2, dv, b2)


def _dot_body(zs_ref, zd_ref, o_ref):
    o_ref[...] = jnp.sum(zs_ref[...] * zd_ref[...], axis=1, keepdims=True)


def _dot(zs, zd):
    EL, DO = zs.shape
    RB = 4096
    grid = EL // RB
    return pl.pallas_call(
        _dot_body,
        grid=(grid,),
        in_specs=[
            pl.BlockSpec((RB, DO), lambda i: (i, 0)),
            pl.BlockSpec((RB, DO), lambda i: (i, 0)),
        ],
        out_specs=pl.BlockSpec((RB, 1), lambda i: (i, 0)),
        out_shape=jax.ShapeDtypeStruct((EL, 1), jnp.float32),
    )(zs, zd)


def kernel(x, edge_index, edge_label_index, W1, b1, W2, b2):
    N = x.shape[0]
    E = edge_index.shape[1]
    EL = edge_label_index.shape[1]
    DH = W1.shape[1]
    DO = W2.shape[1]

    DP = 128
    W2p = jnp.pad(W2, ((0, 0), (0, DP - DO)))
    b2p = jnp.pad(b2, (0, DP - DO))

    ec = jnp.transpose(edge_index.reshape(2, E // 80, 80), (1, 0, 2))
    elc = jnp.transpose(edge_label_index.reshape(2, EL // 64, 64), (1, 0, 2))

    degp = _deg_kernel(E, N, DP)(
        ec, jnp.zeros((N, DP), jnp.float32), jnp.ones((80, DP), jnp.float32))
    y1, dv = _mm1(x, W1, degp)
    s1p = _scatter_kernel(E, N, DH)(y1, ec)
    y2 = _mm2(s1p, y1, dv, b1.reshape(1, -1), W2p)
    s2p = _scatter_kernel(E, N, DP)(y2, ec)
    z = _fin(s2p, y2, dv, b2p.reshape(1, -1))
    zs, zd = _gather_kernel(EL, N, DP)(z, elc)
    return _dot(zs, zd).reshape(-1)


# speedup vs baseline: 23.4407x; 1.0747x over previous
import functools

import jax
import jax.numpy as jnp
from jax import lax
from jax.experimental import pallas as pl
from jax.experimental.pallas import tpu as pltpu
from jax.experimental.pallas import tpu_sc as plsc

NC = 2
NS = 16
LANES = 16
NW = NC * NS

_MESH = plsc.VectorSubcoreMesh(core_axis_name="c", subcore_axis_name="s")


@functools.lru_cache(maxsize=None)
def _deg_kernel(E: int, N: int, D: int):
    per_w = E // NW
    C = 80
    n_ch = per_w // C
    rmain = (N // NS) // 8 * 8
    tbase = rmain * NS
    tail = N - tbase

    @functools.partial(
        pl.kernel,
        out_type=jax.ShapeDtypeStruct((NC, N, D), jnp.float32),
        mesh=_MESH,
        compiler_params=pltpu.CompilerParams(use_tc_tiling_on_sc=False),
        scratch_types=[
            pltpu.VMEM((per_w // C, 2, C), jnp.int32),
            pltpu.VMEM((C, D), jnp.float32),
            pltpu.VMEM_SHARED((N, D), jnp.float32),
            pltpu.SemaphoreType.DMA,
            pltpu.SemaphoreType.DMA,
        ],
    )
    def k(ec_hbm, zeros_hbm, ones_hbm, out_hbm, idx_v, ones_v, acc,
          ssem0, ssem1):
        cid = lax.axis_index("c")
        sid = lax.axis_index("s")
        wid = cid * NS + sid

        pltpu.sync_copy(ones_hbm, ones_v)
        pltpu.sync_copy(ec_hbm.at[pl.ds(wid * n_ch, n_ch)], idx_v)
        r0 = pl.multiple_of(sid * rmain, 8)
        pltpu.sync_copy(zeros_hbm.at[pl.ds(r0, rmain)],
                        acc.at[pl.ds(r0, rmain)])

        @pl.when(sid == 0)
        def _():
            pltpu.sync_copy(zeros_hbm.at[pl.ds(tbase, tail)],
                            acc.at[pl.ds(tbase, tail)])

        plsc.subcore_barrier()

        ssems = (ssem0, ssem1)

        def scat_desc(jj, b):
            return pltpu.make_async_copy(ones_v, acc.at[idx_v.at[jj, 1]],
                                         ssems[b])

        @pl.loop(0, n_ch, step=2)
        def _(j):
            for b in (0, 1):
                jj = j + b

                @pl.when(jj < n_ch)
                def _():
                    @pl.when(jj >= 2)
                    def _():
                        scat_desc(jj - 2, b).wait()

                    pltpu.async_copy(ones_v, acc.at[idx_v.at[jj, 1]],
                                     ssems[b], add=True)

        scat_desc(n_ch - 2, (n_ch - 2) % 2).wait()
        scat_desc(n_ch - 1, (n_ch - 1) % 2).wait()
        plsc.subcore_barrier()
        pltpu.sync_copy(acc.at[pl.ds(r0, rmain)],
                        out_hbm.at[cid, pl.ds(r0, rmain)])

        @pl.when(sid == 0)
        def _():
            pltpu.sync_copy(acc.at[pl.ds(tbase, tail)],
                            out_hbm.at[cid, pl.ds(tbase, tail)])

    return k


@functools.lru_cache(maxsize=None)
def _scatter_kernel(E: int, N: int, D: int):
    per_w = E // NW
    C = 80
    n_ch = per_w // C
    rmain = (N // NS) // 8 * 8
    tbase = rmain * NS
    tail = N - tbase

    half = (n_ch + 1) // 2
    phases = ((0, half), (half, n_ch - half))

    @functools.partial(
        pl.kernel,
        out_type=jax.ShapeDtypeStruct((NC, N, D), jnp.float32),
        mesh=_MESH,
        compiler_params=pltpu.CompilerParams(use_tc_tiling_on_sc=False),
        scratch_types=[
            pltpu.VMEM((half, 2, C), jnp.int32),
            pltpu.VMEM((2, C, D), jnp.float32),
            pltpu.VMEM_SHARED((N, D), jnp.float32),
            pltpu.SemaphoreType.DMA,
            pltpu.SemaphoreType.DMA,
            pltpu.SemaphoreType.DMA,
            pltpu.SemaphoreType.DMA,
        ],
    )
    def k(y_hbm, ec_hbm, out_hbm, idx_v, rows_v, acc,
          gsem0, gsem1, ssem0, ssem1):
        cid = lax.axis_index("c")
        sid = lax.axis_index("s")
        wid = cid * NS + sid

        pltpu.sync_copy(ec_hbm.at[pl.ds(wid * n_ch, half)],
                        idx_v.at[pl.ds(0, half)])
        r0 = pl.multiple_of(sid * rmain, 8)
        pltpu.sync_copy(y_hbm.at[pl.ds(r0, rmain)], acc.at[pl.ds(r0, rmain)])

        @pl.when(sid == 0)
        def _():
            pltpu.sync_copy(y_hbm.at[pl.ds(tbase, tail)],
                            acc.at[pl.ds(tbase, tail)])

        plsc.subcore_barrier()

        gsems = (gsem0, gsem1)
        ssems = (ssem0, ssem1)

        def gath_desc(jj, b):
            return pltpu.make_async_copy(y_hbm.at[idx_v.at[jj, 0]],
                                         rows_v.at[b], gsems[b])

        def scat_desc(jj, b):
            return pltpu.make_async_copy(rows_v.at[b],
                                         acc.at[idx_v.at[jj, 1]], ssems[b])

        for c0, cn in phases:
            if c0 > 0:
                pltpu.sync_copy(ec_hbm.at[pl.ds(wid * n_ch + c0, cn)],
                                idx_v.at[pl.ds(0, cn)])
            pltpu.async_copy(y_hbm.at[idx_v.at[0, 0]], rows_v.at[0], gsem0)

            @pl.loop(0, cn, step=2)
            def _(j):
                for b in (0, 1):
                    jj = j + b
                    nb = 1 - b

                    @pl.when(jj + 1 < cn)
                    def _():
                        @pl.when(jj >= 1)
                        def _():
                            scat_desc(jj - 1, nb).wait()

                        pltpu.async_copy(y_hbm.at[idx_v.at[jj + 1, 0]],
                                         rows_v.at[nb], gsems[nb])

                    @pl.when(jj < cn)
                    def _():
                        gath_desc(jj, b).wait()
                        pltpu.async_copy(rows_v.at[b],
                                         acc.at[idx_v.at[jj, 1]],
                                         ssems[b], add=True)

            scat_desc(cn - 2, (cn - 2) % 2).wait()
            scat_desc(cn - 1, (cn - 1) % 2).wait()

        plsc.subcore_barrier()
        pltpu.sync_copy(acc.at[pl.ds(r0, rmain)],
                        out_hbm.at[cid, pl.ds(r0, rmain)])

        @pl.when(sid == 0)
        def _():
            pltpu.sync_copy(acc.at[pl.ds(tbase, tail)],
                            out_hbm.at[cid, pl.ds(tbase, tail)])

    return k


@functools.lru_cache(maxsize=None)
def _gather_kernel(EL: int, N: int, D: int):
    per_w = EL // NW
    C = 64
    n_ch = per_w // C

    @functools.partial(
        pl.kernel,
        out_type=(jax.ShapeDtypeStruct((EL, D), jnp.float32),
                  jax.ShapeDtypeStruct((EL, D), jnp.float32)),
        mesh=_MESH,
        compiler_params=pltpu.CompilerParams(use_tc_tiling_on_sc=False),
        scratch_types=[
            pltpu.VMEM((per_w // C, 2, C), jnp.int32),
            pltpu.VMEM((2, C, D), jnp.float32),
            pltpu.VMEM((2, C, D), jnp.float32),
            pltpu.SemaphoreType.DMA,
            pltpu.SemaphoreType.DMA,
            pltpu.SemaphoreType.DMA,
            pltpu.SemaphoreType.DMA,
            pltpu.SemaphoreType.DMA,
            pltpu.SemaphoreType.DMA,
            pltpu.SemaphoreType.DMA,
            pltpu.SemaphoreType.DMA,
        ],
    )
    def k(z_hbm, ec_hbm, zs_hbm, zd_hbm, idx_v, ra_v, rb_v,
          sa0, sa1, sb0, sb1, oa0, oa1, ob0, ob1):
        cid = lax.axis_index("c")
        sid = lax.axis_index("s")
        wid = cid * NS + sid

        sas = (sa0, sa1)
        sbs = (sb0, sb1)
        oas = (oa0, oa1)
        obs = (ob0, ob1)

        pltpu.sync_copy(ec_hbm.at[pl.ds(wid * n_ch, n_ch)], idx_v)

        def out_of(j):
            return pl.ds(pl.multiple_of(wid * per_w + j * C, 8), C)

        def gathers(jj, b):
            pltpu.async_copy(z_hbm.at[idx_v.at[jj, 0]], ra_v.at[b], sas[b])
            pltpu.async_copy(z_hbm.at[idx_v.at[jj, 1]], rb_v.at[b], sbs[b])

        def odesc(jj, b):
            return (pltpu.make_async_copy(ra_v.at[b], zs_hbm.at[out_of(jj)],
                                          oas[b]),
                    pltpu.make_async_copy(rb_v.at[b], zd_hbm.at[out_of(jj)],
                                          obs[b]))

        gathers(0, 0)

        @pl.loop(0, n_ch, step=2)
        def _(j):
            for b in (0, 1):
                jj = j + b
                nb = 1 - b

                @pl.when(jj + 1 < n_ch)
                def _():
                    @pl.when(jj >= 1)
                    def _():
                        da, db = odesc(jj - 1, nb)
                        da.wait()
                        db.wait()

                    gathers(jj + 1, nb)

                @pl.when(jj < n_ch)
                def _():
                    pltpu.make_async_copy(z_hbm.at[idx_v.at[jj, 0]],
                                          ra_v.at[b], sas[b]).wait()
                    pltpu.async_copy(ra_v.at[b], zs_hbm.at[out_of(jj)],
                                     oas[b])
                    pltpu.make_async_copy(z_hbm.at[idx_v.at[jj, 1]],
                                          rb_v.at[b], sbs[b]).wait()
                    pltpu.async_copy(rb_v.at[b], zd_hbm.at[out_of(jj)],
                                     obs[b])

        for last in (n_ch - 2, n_ch - 1):
            da, db = odesc(last, last % 2)
            da.wait()
            db.wait()

    return k


_R = 1000


def _mm1_body(x_ref, w_ref, dp_ref, y_ref, dv_ref):
    dp = dp_ref[...]
    deg = dp[0][:, 0:1] + dp[1][:, 0:1] + 1.0
    dinv = lax.rsqrt(deg)
    y = jnp.dot(x_ref[...], w_ref[...], preferred_element_type=jnp.float32)
    y_ref[...] = y * dinv
    dv_ref[...] = jnp.broadcast_to(dinv, dv_ref.shape)


def _mm1(x, W1, degp):
    N, DI = x.shape
    DH = W1.shape[1]
    DG = degp.shape[2]
    grid = N // _R
    return pl.pallas_call(
        _mm1_body,
        grid=(grid,),
        in_specs=[
            pl.BlockSpec((_R, DI), lambda i: (i, 0)),
            pl.BlockSpec((DI, DH), lambda i: (0, 0)),
            pl.BlockSpec((NC, _R, DG), lambda i: (0, i, 0)),
        ],
        out_specs=[
            pl.BlockSpec((_R, DH), lambda i: (i, 0)),
            pl.BlockSpec((_R, LANES), lambda i: (i, 0)),
        ],
        out_shape=[
            jax.ShapeDtypeStruct((N, DH), jnp.float32),
            jax.ShapeDtypeStruct((N, LANES), jnp.float32),
        ],
    )(x, W1, degp)


def _mm2_body(sp_ref, y1_ref, dv_ref, b1_ref, w_ref, y2_ref):
    sp = sp_ref[...]
    y1 = y1_ref[...]
    dinv = dv_ref[...][:, 0:1]
    s = sp[0] + sp[1] - y1
    h = jnp.maximum(s * dinv + b1_ref[...], 0.0)
    y2_ref[...] = jnp.dot(h * dinv, w_ref[...],
                          preferred_element_type=jnp.float32)


def _mm2(s1p, y1, dv, b1, W2):
    N, DH = y1.shape
    DO = W2.shape[1]
    grid = N // _R
    return pl.pallas_call(
        _mm2_body,
        grid=(grid,),
        in_specs=[
            pl.BlockSpec((NC, _R, DH), lambda i: (0, i, 0)),
            pl.BlockSpec((_R, DH), lambda i: (i, 0)),
            pl.BlockSpec((_R, LANES), lambda i: (i, 0)),
            pl.BlockSpec((1, DH), lambda i: (0, 0)),
            pl.BlockSpec((DH, DO), lambda i: (0, 0)),
        ],
        out_specs=pl.BlockSpec((_R, DO), lambda i: (i, 0)),
        out_shape=jax.ShapeDtypeStruct((N, DO), jnp.float32),
    )(s1p, y1, dv, b1, W2)


def _fin_body(sp_ref, y2_ref, dv_ref, b2_ref, z_ref):
    sp = sp_ref[...]
    y2 = y2_ref[...]
    dinv = dv_ref[...][:, 0:1]
    z_ref[...] = (sp[0] + sp[1] - y2) * dinv + b2_ref[...]


def _fin(s2p, y2, dv, b2):
    N, DO = y2.shape
    grid = N // _R
    return pl.pallas_call(
        _fin_body,
        grid=(grid,),
        in_specs=[
            pl.BlockSpec((NC, _R, DO), lambda i: (0, i, 0)),
            pl.BlockSpec((_R, DO), lambda i: (i, 0)),
            pl.BlockSpec((_R, LANES), lambda i: (i, 0)),
            pl.BlockSpec((1, DO), lambda i: (0, 0)),
        ],
        out_specs=pl.BlockSpec((_R, DO), lambda i: (i, 0)),
        out_shape=jax.ShapeDtypeStruct((N, DO), jnp.float32),
    )(s2p, y2, dv, b2)


def _dot_body(zs_ref, zd_ref, o_ref):
    o_ref[...] = jnp.sum(zs_ref[...] * zd_ref[...], axis=1, keepdims=True)


def _dot(zs, zd):
    EL, DO = zs.shape
    RB = 4096
    grid = EL // RB
    return pl.pallas_call(
        _dot_body,
        grid=(grid,),
        in_specs=[
            pl.BlockSpec((RB, DO), lambda i: (i, 0)),
            pl.BlockSpec((RB, DO), lambda i: (i, 0)),
        ],
        out_specs=pl.BlockSpec((RB, 1), lambda i: (i, 0)),
        out_shape=jax.ShapeDtypeStruct((EL, 1), jnp.float32),
    )(zs, zd)


def kernel(x, edge_index, edge_label_index, W1, b1, W2, b2):
    N = x.shape[0]
    E = edge_index.shape[1]
    EL = edge_label_index.shape[1]
    DH = W1.shape[1]
    DO = W2.shape[1]

    ec = jnp.transpose(edge_index.reshape(2, E // 80, 80), (1, 0, 2))
    elc = jnp.transpose(edge_label_index.reshape(2, EL // 64, 64), (1, 0, 2))

    DG = 16
    degp = _deg_kernel(E, N, DG)(
        ec, jnp.zeros((N, DG), jnp.float32), jnp.ones((80, DG), jnp.float32))
    y1, dv = _mm1(x, W1, degp)
    s1p = _scatter_kernel(E, N, DH)(y1, ec)
    y2 = _mm2(s1p, y1, dv, b1.reshape(1, -1), W2)
    s2p = _scatter_kernel(E, N, DO)(y2, ec)
    z = _fin(s2p, y2, dv, b2.reshape(1, -1))
    zs, zd = _gather_kernel(EL, N, DO)(z, elc)
    return _dot(zs, zd).reshape(-1)


# speedup vs baseline: 24.5161x; 1.0459x over previous
import functools

import jax
import jax.numpy as jnp
from jax import lax
from jax.experimental import pallas as pl
from jax.experimental.pallas import tpu as pltpu
from jax.experimental.pallas import tpu_sc as plsc

NC = 2
NS = 16
LANES = 16
NW = NC * NS

_MESH = plsc.VectorSubcoreMesh(core_axis_name="c", subcore_axis_name="s")


@functools.lru_cache(maxsize=None)
def _deg_kernel(E: int, N: int, D: int):
    per_w = E // NW
    C = 80
    n_ch = per_w // C
    rmain = (N // NS) // 8 * 8
    tbase = rmain * NS
    tail = N - tbase

    @functools.partial(
        pl.kernel,
        out_type=jax.ShapeDtypeStruct((NC, N, D), jnp.float32),
        mesh=_MESH,
        compiler_params=pltpu.CompilerParams(use_tc_tiling_on_sc=False),
        scratch_types=[
            pltpu.VMEM((per_w // C, 2, C), jnp.int32),
            pltpu.VMEM((C, D), jnp.float32),
            pltpu.VMEM_SHARED((N, D), jnp.float32),
            pltpu.SemaphoreType.DMA,
            pltpu.SemaphoreType.DMA,
        ],
    )
    def k(ec_hbm, zeros_hbm, ones_hbm, out_hbm, idx_v, ones_v, acc,
          ssem0, ssem1):
        cid = lax.axis_index("c")
        sid = lax.axis_index("s")
        wid = cid * NS + sid

        pltpu.sync_copy(ones_hbm, ones_v)
        pltpu.sync_copy(ec_hbm.at[pl.ds(wid * n_ch, n_ch)], idx_v)
        r0 = pl.multiple_of(sid * rmain, 8)
        pltpu.sync_copy(zeros_hbm.at[pl.ds(r0, rmain)],
                        acc.at[pl.ds(r0, rmain)])

        @pl.when(sid == 0)
        def _():
            pltpu.sync_copy(zeros_hbm.at[pl.ds(tbase, tail)],
                            acc.at[pl.ds(tbase, tail)])

        plsc.subcore_barrier()

        ssems = (ssem0, ssem1)

        def scat_desc(jj, b):
            return pltpu.make_async_copy(ones_v, acc.at[idx_v.at[jj, 1]],
                                         ssems[b])

        @pl.loop(0, n_ch, step=2)
        def _(j):
            for b in (0, 1):
                jj = j + b

                @pl.when(jj < n_ch)
                def _():
                    @pl.when(jj >= 2)
                    def _():
                        scat_desc(jj - 2, b).wait()

                    pltpu.async_copy(ones_v, acc.at[idx_v.at[jj, 1]],
                                     ssems[b], add=True)

        scat_desc(n_ch - 2, (n_ch - 2) % 2).wait()
        scat_desc(n_ch - 1, (n_ch - 1) % 2).wait()
        plsc.subcore_barrier()
        pltpu.sync_copy(acc.at[pl.ds(r0, rmain)],
                        out_hbm.at[cid, pl.ds(r0, rmain)])

        @pl.when(sid == 0)
        def _():
            pltpu.sync_copy(acc.at[pl.ds(tbase, tail)],
                            out_hbm.at[cid, pl.ds(tbase, tail)])

    return k


@functools.lru_cache(maxsize=None)
def _scatter_kernel(E: int, N: int, D: int):
    per_w = E // NW
    C = 80
    n_ch = per_w // C
    rmain = (N // NS) // 8 * 8
    tbase = rmain * NS
    tail = N - tbase

    half = (n_ch + 1) // 2
    phases = ((0, half), (half, n_ch - half))

    @functools.partial(
        pl.kernel,
        out_type=jax.ShapeDtypeStruct((NC, N, D), jnp.float32),
        mesh=_MESH,
        compiler_params=pltpu.CompilerParams(use_tc_tiling_on_sc=False),
        scratch_types=[
            pltpu.VMEM((half, 2, C), jnp.int32),
            pltpu.VMEM((2, C, D), jnp.float32),
            pltpu.VMEM_SHARED((N, D), jnp.float32),
            pltpu.SemaphoreType.DMA,
            pltpu.SemaphoreType.DMA,
            pltpu.SemaphoreType.DMA,
            pltpu.SemaphoreType.DMA,
        ],
    )
    def k(y_hbm, ec_hbm, out_hbm, idx_v, rows_v, acc,
          gsem0, gsem1, ssem0, ssem1):
        cid = lax.axis_index("c")
        sid = lax.axis_index("s")
        wid = cid * NS + sid

        pltpu.sync_copy(ec_hbm.at[pl.ds(wid * n_ch, half)],
                        idx_v.at[pl.ds(0, half)])
        r0 = pl.multiple_of(sid * rmain, 8)
        pltpu.sync_copy(y_hbm.at[pl.ds(r0, rmain)], acc.at[pl.ds(r0, rmain)])

        @pl.when(sid == 0)
        def _():
            pltpu.sync_copy(y_hbm.at[pl.ds(tbase, tail)],
                            acc.at[pl.ds(tbase, tail)])

        plsc.subcore_barrier()

        gsems = (gsem0, gsem1)
        ssems = (ssem0, ssem1)

        def gath_desc(jj, b):
            return pltpu.make_async_copy(y_hbm.at[idx_v.at[jj, 0]],
                                         rows_v.at[b], gsems[b])

        def scat_desc(jj, b):
            return pltpu.make_async_copy(rows_v.at[b],
                                         acc.at[idx_v.at[jj, 1]], ssems[b])

        for c0, cn in phases:
            if c0 > 0:
                pltpu.sync_copy(ec_hbm.at[pl.ds(wid * n_ch + c0, cn)],
                                idx_v.at[pl.ds(0, cn)])
            pltpu.async_copy(y_hbm.at[idx_v.at[0, 0]], rows_v.at[0], gsem0)

            @pl.loop(0, cn, step=2)
            def _(j):
                for b in (0, 1):
                    jj = j + b
                    nb = 1 - b

                    @pl.when(jj + 1 < cn)
                    def _():
                        @pl.when(jj >= 1)
                        def _():
                            scat_desc(jj - 1, nb).wait()

                        pltpu.async_copy(y_hbm.at[idx_v.at[jj + 1, 0]],
                                         rows_v.at[nb], gsems[nb])

                    @pl.when(jj < cn)
                    def _():
                        gath_desc(jj, b).wait()
                        pltpu.async_copy(rows_v.at[b],
                                         acc.at[idx_v.at[jj, 1]],
                                         ssems[b], add=True)

            scat_desc(cn - 2, (cn - 2) % 2).wait()
            scat_desc(cn - 1, (cn - 1) % 2).wait()

        plsc.subcore_barrier()
        pltpu.sync_copy(acc.at[pl.ds(r0, rmain)],
                        out_hbm.at[cid, pl.ds(r0, rmain)])

        @pl.when(sid == 0)
        def _():
            pltpu.sync_copy(acc.at[pl.ds(tbase, tail)],
                            out_hbm.at[cid, pl.ds(tbase, tail)])

    return k


@functools.lru_cache(maxsize=None)
def _gather_dot_kernel(EL: int, N: int, D: int):
    per_w = EL // NW
    C = 64
    n_ch = per_w // C
    G = C // LANES

    @functools.partial(
        pl.kernel,
        out_type=jax.ShapeDtypeStruct((EL, LANES), jnp.float32),
        mesh=_MESH,
        compiler_params=pltpu.CompilerParams(use_tc_tiling_on_sc=False),
        scratch_types=[
            pltpu.VMEM((per_w // C, 2, C), jnp.int32),
            pltpu.VMEM((2, C, D), jnp.float32),
            pltpu.VMEM((2, C, D), jnp.float32),
            pltpu.VMEM((2, C, LANES), jnp.float32),
            pltpu.SemaphoreType.DMA,
            pltpu.SemaphoreType.DMA,
            pltpu.SemaphoreType.DMA,
            pltpu.SemaphoreType.DMA,
            pltpu.SemaphoreType.DMA,
            pltpu.SemaphoreType.DMA,
        ],
    )
    def k(z_hbm, ec_hbm, out_hbm, idx_v, ra_v, rb_v, sc_v,
          sa0, sa1, sb0, sb1, os0, os1):
        cid = lax.axis_index("c")
        sid = lax.axis_index("s")
        wid = cid * NS + sid

        sas = (sa0, sa1)
        sbs = (sb0, sb1)
        oss = (os0, os1)

        pltpu.sync_copy(ec_hbm.at[pl.ds(wid * n_ch, n_ch)], idx_v)

        def out_of(j):
            return pl.ds(pl.multiple_of(wid * per_w + j * C, 8), C)

        def gathers(jj, b):
            pltpu.async_copy(z_hbm.at[idx_v.at[jj, 0]], ra_v.at[b], sas[b])
            pltpu.async_copy(z_hbm.at[idx_v.at[jj, 1]], rb_v.at[b], sbs[b])

        def odesc(jj, b):
            return pltpu.make_async_copy(sc_v.at[b], out_hbm.at[out_of(jj)],
                                         oss[b])

        gathers(0, 0)

        @pl.loop(0, n_ch, step=2)
        def _(j):
            for b in (0, 1):
                jj = j + b
                nb = 1 - b

                @pl.when(jj + 1 < n_ch)
                def _():
                    gathers(jj + 1, nb)

                @pl.when(jj < n_ch)
                def _():
                    pltpu.make_async_copy(z_hbm.at[idx_v.at[jj, 0]],
                                          ra_v.at[b], sas[b]).wait()
                    pltpu.make_async_copy(z_hbm.at[idx_v.at[jj, 1]],
                                          rb_v.at[b], sbs[b]).wait()

                    @pl.when(jj >= 2)
                    def _():
                        odesc(jj - 2, b).wait()

                    for e in range(C):
                        acc = ra_v[b, e, 0:LANES] * rb_v[b, e, 0:LANES]
                        for v in range(1, D // LANES):
                            lo = v * LANES
                            acc = acc + (ra_v[b, e, lo:lo + LANES]
                                         * rb_v[b, e, lo:lo + LANES])
                        sc_v[b, e, :] = acc

                    pltpu.async_copy(sc_v.at[b], out_hbm.at[out_of(jj)],
                                     oss[b])

        for last in (n_ch - 2, n_ch - 1):
            odesc(last, last % 2).wait()

    return k


_R = 1000


def _mm1_body(x_ref, w_ref, dp_ref, y_ref, dv_ref):
    dp = dp_ref[...]
    deg = dp[0][:, 0:1] + dp[1][:, 0:1] + 1.0
    dinv = lax.rsqrt(deg)
    y = jnp.dot(x_ref[...], w_ref[...], preferred_element_type=jnp.float32)
    y_ref[...] = y * dinv
    dv_ref[...] = jnp.broadcast_to(dinv, dv_ref.shape)


def _mm1(x, W1, degp):
    N, DI = x.shape
    DH = W1.shape[1]
    DG = degp.shape[2]
    grid = N // _R
    return pl.pallas_call(
        _mm1_body,
        grid=(grid,),
        in_specs=[
            pl.BlockSpec((_R, DI), lambda i: (i, 0)),
            pl.BlockSpec((DI, DH), lambda i: (0, 0)),
            pl.BlockSpec((NC, _R, DG), lambda i: (0, i, 0)),
        ],
        out_specs=[
            pl.BlockSpec((_R, DH), lambda i: (i, 0)),
            pl.BlockSpec((_R, LANES), lambda i: (i, 0)),
        ],
        out_shape=[
            jax.ShapeDtypeStruct((N, DH), jnp.float32),
            jax.ShapeDtypeStruct((N, LANES), jnp.float32),
        ],
    )(x, W1, degp)


def _mm2_body(sp_ref, y1_ref, dv_ref, b1_ref, w_ref, y2_ref):
    sp = sp_ref[...]
    y1 = y1_ref[...]
    dinv = dv_ref[...][:, 0:1]
    s = sp[0] + sp[1] - y1
    h = jnp.maximum(s * dinv + b1_ref[...], 0.0)
    y2_ref[...] = jnp.dot(h * dinv, w_ref[...],
                          preferred_element_type=jnp.float32)


def _mm2(s1p, y1, dv, b1, W2):
    N, DH = y1.shape
    DO = W2.shape[1]
    grid = N // _R
    return pl.pallas_call(
        _mm2_body,
        grid=(grid,),
        in_specs=[
            pl.BlockSpec((NC, _R, DH), lambda i: (0, i, 0)),
            pl.BlockSpec((_R, DH), lambda i: (i, 0)),
            pl.BlockSpec((_R, LANES), lambda i: (i, 0)),
            pl.BlockSpec((1, DH), lambda i: (0, 0)),
            pl.BlockSpec((DH, DO), lambda i: (0, 0)),
        ],
        out_specs=pl.BlockSpec((_R, DO), lambda i: (i, 0)),
        out_shape=jax.ShapeDtypeStruct((N, DO), jnp.float32),
    )(s1p, y1, dv, b1, W2)


def _fin_body(sp_ref, y2_ref, dv_ref, b2_ref, z_ref):
    sp = sp_ref[...]
    y2 = y2_ref[...]
    dinv = dv_ref[...][:, 0:1]
    z_ref[...] = (sp[0] + sp[1] - y2) * dinv + b2_ref[...]


def _fin(s2p, y2, dv, b2):
    N, DO = y2.shape
    grid = N // _R
    return pl.pallas_call(
        _fin_body,
        grid=(grid,),
        in_specs=[
            pl.BlockSpec((NC, _R, DO), lambda i: (0, i, 0)),
            pl.BlockSpec((_R, DO), lambda i: (i, 0)),
            pl.BlockSpec((_R, LANES), lambda i: (i, 0)),
            pl.BlockSpec((1, DO), lambda i: (0, 0)),
        ],
        out_specs=pl.BlockSpec((_R, DO), lambda i: (i, 0)),
        out_shape=jax.ShapeDtypeStruct((N, DO), jnp.float32),
    )(s2p, y2, dv, b2)


def _dotsum_body(p_ref, o_ref):
    o_ref[...] = jnp.sum(p_ref[...], axis=1, keepdims=True)


def _dotsum(p):
    EL = p.shape[0]
    RB = 8192
    grid = EL // RB
    return pl.pallas_call(
        _dotsum_body,
        grid=(grid,),
        in_specs=[pl.BlockSpec((RB, LANES), lambda i: (i, 0))],
        out_specs=pl.BlockSpec((RB, 1), lambda i: (i, 0)),
        out_shape=jax.ShapeDtypeStruct((EL, 1), jnp.float32),
    )(p)


def kernel(x, edge_index, edge_label_index, W1, b1, W2, b2):
    N = x.shape[0]
    E = edge_index.shape[1]
    EL = edge_label_index.shape[1]
    DH = W1.shape[1]
    DO = W2.shape[1]

    ec = jnp.transpose(edge_index.reshape(2, E // 80, 80), (1, 0, 2))
    elc = jnp.transpose(edge_label_index.reshape(2, EL // 64, 64), (1, 0, 2))

    DG = 16
    degp = _deg_kernel(E, N, DG)(
        ec, jnp.zeros((N, DG), jnp.float32), jnp.ones((80, DG), jnp.float32))
    y1, dv = _mm1(x, W1, degp)
    s1p = _scatter_kernel(E, N, DH)(y1, ec)
    y2 = _mm2(s1p, y1, dv, b1.reshape(1, -1), W2)
    s2p = _scatter_kernel(E, N, DO)(y2, ec)
    z = _fin(s2p, y2, dv, b2.reshape(1, -1))
    p16 = _gather_dot_kernel(EL, N, DO)(z, elc)
    return _dotsum(p16).reshape(-1)
